# baseline (speedup 1.0000x reference)
import functools

import jax
import jax.numpy as jnp
from jax.experimental import pallas as pl

N = 10000
D = 128
AVG_D_LOG = 3.5

_ROW_BLK = 1000


def _mm_kernel(x_ref, w_ref, b_ref, o_ref, *, relu):
    acc = jnp.dot(x_ref[...], w_ref[...], preferred_element_type=jnp.float32)
    acc = acc + b_ref[...][None, :]
    if relu:
        acc = jnp.maximum(acc, 0.0)
    o_ref[...] = acc


def _matmul_bias(x, w, b, relu=False):
    n, k = x.shape
    m = w.shape[1]
    grid = (n // _ROW_BLK,)
    return pl.pallas_call(
        functools.partial(_mm_kernel, relu=relu),
        grid=grid,
        in_specs=[
            pl.BlockSpec((_ROW_BLK, k), lambda i: (i, 0)),
            pl.BlockSpec((k, m), lambda i: (0, 0)),
            pl.BlockSpec((m,), lambda i: (0,)),
        ],
        out_specs=pl.BlockSpec((_ROW_BLK, m), lambda i: (i, 0)),
        out_shape=jax.ShapeDtypeStruct((n, m), jnp.float32),
    )(x, w, b)


def _combine_kernel(h_ref, s_ref, mx_ref, mn_ref, sq_ref, deg_ref, w_ref,
                    b_ref, o_ref):
    deg = deg_ref[...]
    degc = jnp.maximum(deg, 1.0)
    invd = 1.0 / degc
    has = deg > 0.0
    mean = s_ref[...] * invd
    msq = sq_ref[...] * invd
    std = jnp.sqrt(jnp.maximum(msq - mean * mean, 0.0) + 1e-5)
    mx = jnp.where(has, mx_ref[...], 0.0)
    mn = jnp.where(has, mn_ref[...], 0.0)
    agg = jnp.concatenate([mean, mx, mn, std], axis=1)
    ld = jnp.log(deg + 1.0)
    amp = ld / AVG_D_LOG
    att = AVG_D_LOG / jnp.maximum(ld, 1e-5)
    w = w_ref[...]
    acc = jnp.dot(h_ref[...], w[0:D], preferred_element_type=jnp.float32)
    acc += jnp.dot(agg, w[D:D + 512], preferred_element_type=jnp.float32)
    acc += jnp.dot(agg * amp, w[D + 512:D + 1024],
                   preferred_element_type=jnp.float32)
    acc += jnp.dot(agg * att, w[D + 1024:D + 1536],
                   preferred_element_type=jnp.float32)
    acc += b_ref[...][None, :]
    o_ref[...] = jnp.maximum(acc, 0.0)


def _layer_combine(h, s, mx, mn, sq, deg, w, b):
    grid = (N // _ROW_BLK,)
    blk = lambda i: (i, 0)
    return pl.pallas_call(
        _combine_kernel,
        grid=grid,
        in_specs=[
            pl.BlockSpec((_ROW_BLK, D), blk),
            pl.BlockSpec((_ROW_BLK, D), blk),
            pl.BlockSpec((_ROW_BLK, D), blk),
            pl.BlockSpec((_ROW_BLK, D), blk),
            pl.BlockSpec((_ROW_BLK, D), blk),
            pl.BlockSpec((_ROW_BLK, 1), blk),
            pl.BlockSpec((13 * D, D), lambda i: (0, 0)),
            pl.BlockSpec((D,), lambda i: (0,)),
        ],
        out_specs=pl.BlockSpec((_ROW_BLK, D), blk),
        out_shape=jax.ShapeDtypeStruct((N, D), jnp.float32),
    )(h, s, mx, mn, sq, deg, w, b)


def _aggregate(h, src, dst):
    m = h[src]
    s = jax.ops.segment_sum(m, dst, num_segments=N)
    sq = jax.ops.segment_sum(m * m, dst, num_segments=N)
    mx = jax.ops.segment_max(m, dst, num_segments=N)
    mn = -jax.ops.segment_max(-m, dst, num_segments=N)
    return s, mx, mn, sq


def kernel(h, edge_index, e, W_enc, b_enc, W0, b0, W1, b1, W2, b2, W3, b3,
           W_ro, b_ro):
    src = edge_index[0]
    dst = edge_index[1]
    order = jnp.argsort(dst)
    src = src[order]
    dst = dst[order]
    offs = jnp.searchsorted(dst, jnp.arange(0, 10241, 320, dtype=jnp.int32))
    src = src + (offs[0] - offs[0]).astype(src.dtype)
    deg = jax.ops.segment_sum(jnp.ones((src.shape[0],), jnp.float32), dst,
                              num_segments=N)
    deg2 = deg[:, None]
    h = _matmul_bias(h, W_enc, b_enc)
    for W, b in ((W0, b0), (W1, b1), (W2, b2), (W3, b3)):
        s, mx, mn, sq = _aggregate(h, src, dst)
        mx = jnp.where(deg2 > 0, mx, 0.0)
        mn = jnp.where(deg2 > 0, mn, 0.0)
        h = _layer_combine(h, s, mx, mn, sq, deg2, W, b)
    return _matmul_bias(h, W_ro, b_ro)


# speedup vs baseline: 2.1843x; 2.1843x over previous
import functools

import jax
import jax.numpy as jnp
from jax import lax
from jax.experimental import pallas as pl
from jax.experimental.pallas import tpu as pltpu
from jax.experimental.pallas import tpu_sc as plsc

N = 10000
D = 128
E = 320000
AVG_D_LOG = 3.5

_ROW_BLK = 1000

NPT = 160
NW = 32
NWAVES = 2
NSLOT = NW * NWAVES
NPAD = NPT * NSLOT
GARB = NPT
ARWS = 168
C = 128
NEG = -3.0e38
POS = 3.0e38


def _mm_kernel(x_ref, w_ref, b_ref, o_ref, *, relu):
    acc = jnp.dot(x_ref[...], w_ref[...], preferred_element_type=jnp.float32)
    acc = acc + b_ref[...][None, :]
    if relu:
        acc = jnp.maximum(acc, 0.0)
    o_ref[...] = acc


def _matmul_bias(x, w, b, relu=False):
    n, k = x.shape
    m = w.shape[1]
    grid = (n // _ROW_BLK,)
    return pl.pallas_call(
        functools.partial(_mm_kernel, relu=relu),
        grid=grid,
        in_specs=[
            pl.BlockSpec((_ROW_BLK, k), lambda i: (i, 0)),
            pl.BlockSpec((k, m), lambda i: (0, 0)),
            pl.BlockSpec((m,), lambda i: (0,)),
        ],
        out_specs=pl.BlockSpec((_ROW_BLK, m), lambda i: (i, 0)),
        out_shape=jax.ShapeDtypeStruct((n, m), jnp.float32),
    )(x, w, b)


def _combine_kernel(h_ref, s_ref, mx_ref, mn_ref, sq_ref, deg_ref, w_ref,
                    b_ref, o_ref):
    deg = deg_ref[...]
    degc = jnp.maximum(deg, 1.0)
    invd = 1.0 / degc
    has = deg > 0.0
    mean = s_ref[...] * invd
    msq = sq_ref[...] * invd
    std = jnp.sqrt(jnp.maximum(msq - mean * mean, 0.0) + 1e-5)
    mx = jnp.where(has, mx_ref[...], 0.0)
    mn = jnp.where(has, mn_ref[...], 0.0)
    agg = jnp.concatenate([mean, mx, mn, std], axis=1)
    ld = jnp.log(deg + 1.0)
    amp = ld / AVG_D_LOG
    att = AVG_D_LOG / jnp.maximum(ld, 1e-5)
    w = w_ref[...]
    acc = jnp.dot(h_ref[...], w[0:D], preferred_element_type=jnp.float32)
    acc += jnp.dot(agg, w[D:D + 512], preferred_element_type=jnp.float32)
    acc += jnp.dot(agg * amp, w[D + 512:D + 1024],
                   preferred_element_type=jnp.float32)
    acc += jnp.dot(agg * att, w[D + 1024:D + 1536],
                   preferred_element_type=jnp.float32)
    acc += b_ref[...][None, :]
    o_ref[...] = jnp.maximum(acc, 0.0)


def _layer_combine(h, s, mx, mn, sq, deg, w, b):
    grid = (N // _ROW_BLK,)
    blk = lambda i: (i, 0)
    return pl.pallas_call(
        _combine_kernel,
        grid=grid,
        in_specs=[
            pl.BlockSpec((_ROW_BLK, D), blk),
            pl.BlockSpec((_ROW_BLK, D), blk),
            pl.BlockSpec((_ROW_BLK, D), blk),
            pl.BlockSpec((_ROW_BLK, D), blk),
            pl.BlockSpec((_ROW_BLK, D), blk),
            pl.BlockSpec((_ROW_BLK, 1), blk),
            pl.BlockSpec((13 * D, D), lambda i: (0, 0)),
            pl.BlockSpec((D,), lambda i: (0,)),
        ],
        out_specs=pl.BlockSpec((_ROW_BLK, D), blk),
        out_shape=jax.ShapeDtypeStruct((N, D), jnp.float32),
    )(h, s, mx, mn, sq, deg, w, b)


def _sc_agg_body(h_hbm, src_hbm, dst_hbm, offs_hbm,
                 o_s, o_mx, o_mn, o_sq, o_deg,
                 mx_v, mn_v, deg_v, rows_v, sq_v,
                 srcv, dloc, dglb, offs_v,
                 s_sh, sq_sh, sem):
    cid = lax.axis_index("c")
    sid = lax.axis_index("s")
    wid = sid * 2 + cid
    shbase = sid * ARWS

    iota = lax.iota(jnp.int32, 16)
    zeros16 = jnp.zeros((16,), jnp.float32)
    negv = jnp.full((16,), NEG, jnp.float32)
    posv = jnp.full((16,), POS, jnp.float32)
    ones16 = jnp.ones((16,), jnp.float32)
    cols = [iota + 16 * k for k in range(D // 16)]

    pltpu.sync_copy(offs_hbm, offs_v)

    for wave in range(NWAVES):
        slot = wave * NW + wid
        base_node = slot * NPT

        def _init(i, _):
            for k in range(D // 16):
                mx_v[i, pl.ds(16 * k, 16)] = negv
                mn_v[i, pl.ds(16 * k, 16)] = posv
            deg_v[pl.ds(16 * i, 16)] = zeros16
            return 0
        lax.fori_loop(0, ARWS, _init, 0)

        def _zrow(i, _):
            for k in range(D // 16):
                rows_v[i, pl.ds(16 * k, 16)] = zeros16
            return 0
        lax.fori_loop(0, C, _zrow, 0)
        for sh in (s_sh, sq_sh):
            pltpu.sync_copy(rows_v, sh.at[pl.ds(shbase, C)])
            pltpu.sync_copy(rows_v.at[pl.ds(0, ARWS - C)],
                            sh.at[pl.ds(shbase + C, ARWS - C)])

        sltv = jnp.full((16,), slot, jnp.int32)
        start = lax.reduce_max(plsc.load_gather(offs_v, [sltv]), (0,))
        end = lax.reduce_max(plsc.load_gather(offs_v, [sltv + 1]), (0,))
        astart = start & ~7
        nchunks = (end - astart + (C - 1)) >> 7

        def _chunk(c, _):
            cbase = pl.multiple_of(astart + c * C, 8)
            pltpu.sync_copy(src_hbm.at[pl.ds(cbase, C)], srcv)
            pltpu.sync_copy(dst_hbm.at[pl.ds(cbase, C)], dloc)
            gat = pltpu.async_copy(h_hbm.at[srcv], rows_v, sem)
            for g in range(C // 16):
                ids = jnp.full((16,), cbase + g * 16, jnp.int32) + iota
                valid = (ids >= start) & (ids < end)
                loc = dloc[pl.ds(g * 16, 16)] - base_node
                loc = jnp.where(valid, loc, GARB)
                dloc[pl.ds(g * 16, 16)] = loc
                dglb[pl.ds(g * 16, 16)] = loc + shbase
            gat.wait()

            def _edge(j, _):
                bvec = plsc.load_gather(dloc, [jnp.full((16,), j, jnp.int32)])
                dgi = bvec * 16 + iota
                dg = plsc.load_gather(deg_v, [dgi])
                plsc.store_scatter(deg_v, [dgi], dg + ones16)
                for k in range(D // 16):
                    m = rows_v[j, pl.ds(16 * k, 16)]
                    sq_v[j, pl.ds(16 * k, 16)] = m * m
                    cmx = plsc.load_gather(mx_v, [bvec, cols[k]])
                    plsc.store_scatter(mx_v, [bvec, cols[k]],
                                       jnp.maximum(cmx, m))
                    cmn = plsc.load_gather(mn_v, [bvec, cols[k]])
                    plsc.store_scatter(mn_v, [bvec, cols[k]],
                                       jnp.minimum(cmn, m))
                return 0
            lax.fori_loop(0, C, _edge, 0)

            pltpu.sync_copy(rows_v, s_sh.at[dglb], add=True)
            pltpu.sync_copy(sq_v, sq_sh.at[dglb], add=True)
            return 0

        lax.fori_loop(0, nchunks, _chunk, 0)

        pltpu.sync_copy(mx_v.at[pl.ds(0, NPT)],
                        o_mx.at[pl.ds(base_node, NPT)])
        pltpu.sync_copy(mn_v.at[pl.ds(0, NPT)],
                        o_mn.at[pl.ds(base_node, NPT)])
        pltpu.sync_copy(deg_v.at[pl.ds(0, NPT * 16)],
                        o_deg.at[pl.ds(base_node * 16, NPT * 16)])
        pltpu.sync_copy(s_sh.at[pl.ds(shbase, NPT)],
                        o_s.at[pl.ds(base_node, NPT)])
        pltpu.sync_copy(sq_sh.at[pl.ds(shbase, NPT)],
                        o_sq.at[pl.ds(base_node, NPT)])


def _sc_aggregate(h, src_pad, dst_pad, offs):
    mesh = plsc.VectorSubcoreMesh(core_axis_name="c", subcore_axis_name="s")
    f32 = jnp.float32
    out_type = [
        jax.ShapeDtypeStruct((NPAD, D), f32),
        jax.ShapeDtypeStruct((NPAD, D), f32),
        jax.ShapeDtypeStruct((NPAD, D), f32),
        jax.ShapeDtypeStruct((NPAD, D), f32),
        jax.ShapeDtypeStruct((NPAD * 16,), f32),
    ]
    scratch = [
        pltpu.VMEM((ARWS, D), f32),
        pltpu.VMEM((ARWS, D), f32),
        pltpu.VMEM((ARWS * 16,), f32),
        pltpu.VMEM((C, D), f32),
        pltpu.VMEM((C, D), f32),
        pltpu.VMEM((C,), jnp.int32),
        pltpu.VMEM((C,), jnp.int32),
        pltpu.VMEM((C,), jnp.int32),
        pltpu.VMEM((NSLOT + 8,), jnp.int32),
        pltpu.VMEM_SHARED((16 * ARWS, D), f32),
        pltpu.VMEM_SHARED((16 * ARWS, D), f32),
        pltpu.SemaphoreType.DMA,
    ]
    kern = pl.kernel(_sc_agg_body, out_type=out_type, mesh=mesh,
                     scratch_types=scratch,
                     compiler_params=pltpu.CompilerParams(
                         needs_layout_passes=False))
    return kern(h, src_pad, dst_pad, offs)


def kernel(h, edge_index, e, W_enc, b_enc, W0, b0, W1, b1, W2, b2, W3, b3,
           W_ro, b_ro):
    src = edge_index[0]
    dst = edge_index[1]
    order = jnp.argsort(dst)
    src_s = src[order].astype(jnp.int32)
    dst_s = dst[order].astype(jnp.int32)
    pad = jnp.zeros((C,), jnp.int32)
    src_pad = jnp.concatenate([src_s, pad])
    dst_pad = jnp.concatenate([dst_s, pad])
    bounds = jnp.arange(0, NPAD + 1, NPT, dtype=jnp.int32)
    offs = jnp.searchsorted(dst_s, bounds).astype(jnp.int32)
    offs = jnp.concatenate([offs, jnp.full((7,), E, jnp.int32)])

    h = _matmul_bias(h, W_enc, b_enc)
    for W, b in ((W0, b0), (W1, b1), (W2, b2), (W3, b3)):
        s, mx, mn, sq, deg16 = _sc_aggregate(h, src_pad, dst_pad, offs)
        deg = deg16.reshape(NPAD, 16)[:N, 0:1]
        h = _layer_combine(h, s[:N], mx[:N], mn[:N], sq[:N], deg, W, b)
    return _matmul_bias(h, W_ro, b_ro)


# speedup vs baseline: 4.8337x; 2.2130x over previous
import functools

import jax
import jax.numpy as jnp
from jax import lax
from jax.experimental import pallas as pl
from jax.experimental.pallas import tpu as pltpu
from jax.experimental.pallas import tpu_sc as plsc

N = 10000
D = 128
E = 320000
AVG_D_LOG = 3.5

_ROW_BLK = 1000

NPT = 160
NW = 32
NWAVES = 2
NSLOT = NW * NWAVES
NPAD = NPT * NSLOT
GARB = NPT
ARWS = 168
C = 128
NEG = -3.0e38
POS = 3.0e38


def _mm_kernel(x_ref, w_ref, b_ref, o_ref, *, relu):
    acc = jnp.dot(x_ref[...], w_ref[...], preferred_element_type=jnp.float32)
    acc = acc + b_ref[...][None, :]
    if relu:
        acc = jnp.maximum(acc, 0.0)
    o_ref[...] = acc


def _matmul_bias(x, w, b, relu=False):
    n, k = x.shape
    m = w.shape[1]
    grid = (n // _ROW_BLK,)
    return pl.pallas_call(
        functools.partial(_mm_kernel, relu=relu),
        grid=grid,
        in_specs=[
            pl.BlockSpec((_ROW_BLK, k), lambda i: (i, 0)),
            pl.BlockSpec((k, m), lambda i: (0, 0)),
            pl.BlockSpec((m,), lambda i: (0,)),
        ],
        out_specs=pl.BlockSpec((_ROW_BLK, m), lambda i: (i, 0)),
        out_shape=jax.ShapeDtypeStruct((n, m), jnp.float32),
    )(x, w, b)


def _combine_kernel(h_ref, s_ref, mx_ref, mn_ref, sq_ref, deg_ref, w_ref,
                    b_ref, o_ref):
    deg = deg_ref[...]
    degc = jnp.maximum(deg, 1.0)
    invd = 1.0 / degc
    has = deg > 0.0
    mean = s_ref[...] * invd
    msq = sq_ref[...] * invd
    std = jnp.sqrt(jnp.maximum(msq - mean * mean, 0.0) + 1e-5)
    mx = jnp.where(has, mx_ref[...], 0.0)
    mn = jnp.where(has, mn_ref[...], 0.0)
    agg = jnp.concatenate([mean, mx, mn, std], axis=1)
    ld = jnp.log(deg + 1.0)
    amp = ld / AVG_D_LOG
    att = AVG_D_LOG / jnp.maximum(ld, 1e-5)
    w = w_ref[...]
    acc = jnp.dot(h_ref[...], w[0:D], preferred_element_type=jnp.float32)
    acc += jnp.dot(agg, w[D:D + 512], preferred_element_type=jnp.float32)
    acc += jnp.dot(agg * amp, w[D + 512:D + 1024],
                   preferred_element_type=jnp.float32)
    acc += jnp.dot(agg * att, w[D + 1024:D + 1536],
                   preferred_element_type=jnp.float32)
    acc += b_ref[...][None, :]
    o_ref[...] = jnp.maximum(acc, 0.0)


def _layer_combine(h, s, mx, mn, sq, deg, w, b):
    grid = (N // _ROW_BLK,)
    blk = lambda i: (i, 0)
    return pl.pallas_call(
        _combine_kernel,
        grid=grid,
        in_specs=[
            pl.BlockSpec((_ROW_BLK, D), blk),
            pl.BlockSpec((_ROW_BLK, D), blk),
            pl.BlockSpec((_ROW_BLK, D), blk),
            pl.BlockSpec((_ROW_BLK, D), blk),
            pl.BlockSpec((_ROW_BLK, D), blk),
            pl.BlockSpec((_ROW_BLK, 1), blk),
            pl.BlockSpec((13 * D, D), lambda i: (0, 0)),
            pl.BlockSpec((D,), lambda i: (0,)),
        ],
        out_specs=pl.BlockSpec((_ROW_BLK, D), blk),
        out_shape=jax.ShapeDtypeStruct((N, D), jnp.float32),
    )(h, s, mx, mn, sq, deg, w, b)


def _sc_agg_body(h_hbm, src_hbm, dst_hbm, offs_hbm,
                 o_s, o_mx, o_mn, o_sq, o_deg,
                 mx_v, mn_v, deg_v, rows_v, sq_v,
                 srcv, dloc, dglb, offs_v,
                 s_sh, sq_sh, sem):
    cid = lax.axis_index("c")
    sid = lax.axis_index("s")
    wid = sid * 2 + cid
    shbase = sid * ARWS

    iota = lax.iota(jnp.int32, 16)
    zeros16 = jnp.zeros((16,), jnp.float32)
    negv = jnp.full((16,), NEG, jnp.float32)
    posv = jnp.full((16,), POS, jnp.float32)
    ones16 = jnp.ones((16,), jnp.float32)
    cols = [iota + 16 * k for k in range(D // 16)]

    pltpu.sync_copy(offs_hbm, offs_v)

    for wave in range(NWAVES):
        slot = wave * NW + wid
        base_node = slot * NPT

        def _init(i, _):
            for k in range(D // 16):
                mx_v[i, pl.ds(16 * k, 16)] = negv
                mn_v[i, pl.ds(16 * k, 16)] = posv
            deg_v[pl.ds(16 * i, 16)] = zeros16
            return 0
        lax.fori_loop(0, ARWS, _init, 0)

        def _zrow(i, _):
            for k in range(D // 16):
                rows_v[i, pl.ds(16 * k, 16)] = zeros16
            return 0
        lax.fori_loop(0, C, _zrow, 0)
        for sh in (s_sh, sq_sh):
            pltpu.sync_copy(rows_v, sh.at[pl.ds(shbase, C)])
            pltpu.sync_copy(rows_v.at[pl.ds(0, ARWS - C)],
                            sh.at[pl.ds(shbase + C, ARWS - C)])

        sltv = jnp.full((16,), slot, jnp.int32)
        start = lax.reduce_max(plsc.load_gather(offs_v, [sltv]), (0,))
        end = lax.reduce_max(plsc.load_gather(offs_v, [sltv + 1]), (0,))
        astart = start & ~7
        nchunks = (end - astart + (C - 1)) >> 7

        def _chunk(c, _):
            cbase = pl.multiple_of(astart + c * C, 8)
            pltpu.sync_copy(src_hbm.at[pl.ds(cbase, C)], srcv)
            pltpu.sync_copy(dst_hbm.at[pl.ds(cbase, C)], dloc)
            gat = pltpu.async_copy(h_hbm.at[srcv], rows_v, sem)
            for g in range(C // 16):
                ids = jnp.full((16,), cbase + g * 16, jnp.int32) + iota
                valid = (ids >= start) & (ids < end)
                loc = dloc[pl.ds(g * 16, 16)] - base_node
                loc = jnp.where(valid, loc, GARB)
                dloc[pl.ds(g * 16, 16)] = loc
                dglb[pl.ds(g * 16, 16)] = loc + shbase
            gat.wait()

            def _edge(j, _):
                bvec = plsc.load_gather(dloc, [jnp.full((16,), j, jnp.int32)])
                dgi = bvec * 16 + iota
                ms = [rows_v[j, pl.ds(16 * k, 16)] for k in range(D // 16)]
                cmx = [plsc.load_gather(mx_v, [bvec, cols[k]])
                       for k in range(D // 16)]
                cmn = [plsc.load_gather(mn_v, [bvec, cols[k]])
                       for k in range(D // 16)]
                dg = plsc.load_gather(deg_v, [dgi])
                for k in range(D // 16):
                    sq_v[j, pl.ds(16 * k, 16)] = ms[k] * ms[k]
                for k in range(D // 16):
                    plsc.store_scatter(mx_v, [bvec, cols[k]],
                                       jnp.maximum(cmx[k], ms[k]))
                    plsc.store_scatter(mn_v, [bvec, cols[k]],
                                       jnp.minimum(cmn[k], ms[k]))
                plsc.store_scatter(deg_v, [dgi], dg + ones16)
                return 0
            lax.fori_loop(0, C, _edge, 0)

            pltpu.sync_copy(rows_v, s_sh.at[dglb], add=True)
            pltpu.sync_copy(sq_v, sq_sh.at[dglb], add=True)
            return 0

        lax.fori_loop(0, nchunks, _chunk, 0)

        pltpu.sync_copy(mx_v.at[pl.ds(0, NPT)],
                        o_mx.at[pl.ds(base_node, NPT)])
        pltpu.sync_copy(mn_v.at[pl.ds(0, NPT)],
                        o_mn.at[pl.ds(base_node, NPT)])
        pltpu.sync_copy(deg_v.at[pl.ds(0, NPT * 16)],
                        o_deg.at[pl.ds(base_node * 16, NPT * 16)])
        pltpu.sync_copy(s_sh.at[pl.ds(shbase, NPT)],
                        o_s.at[pl.ds(base_node, NPT)])
        pltpu.sync_copy(sq_sh.at[pl.ds(shbase, NPT)],
                        o_sq.at[pl.ds(base_node, NPT)])


def _sc_aggregate(h, src_pad, dst_pad, offs):
    mesh = plsc.VectorSubcoreMesh(core_axis_name="c", subcore_axis_name="s")
    f32 = jnp.float32
    out_type = [
        jax.ShapeDtypeStruct((NPAD, D), f32),
        jax.ShapeDtypeStruct((NPAD, D), f32),
        jax.ShapeDtypeStruct((NPAD, D), f32),
        jax.ShapeDtypeStruct((NPAD, D), f32),
        jax.ShapeDtypeStruct((NPAD * 16,), f32),
    ]
    scratch = [
        pltpu.VMEM((ARWS, D), f32),
        pltpu.VMEM((ARWS, D), f32),
        pltpu.VMEM((ARWS * 16,), f32),
        pltpu.VMEM((C, D), f32),
        pltpu.VMEM((C, D), f32),
        pltpu.VMEM((C,), jnp.int32),
        pltpu.VMEM((C,), jnp.int32),
        pltpu.VMEM((C,), jnp.int32),
        pltpu.VMEM((NSLOT + 8,), jnp.int32),
        pltpu.VMEM_SHARED((16 * ARWS, D), f32),
        pltpu.VMEM_SHARED((16 * ARWS, D), f32),
        pltpu.SemaphoreType.DMA,
    ]
    kern = pl.kernel(_sc_agg_body, out_type=out_type, mesh=mesh,
                     scratch_types=scratch,
                     compiler_params=pltpu.CompilerParams(
                         needs_layout_passes=False))
    return kern(h, src_pad, dst_pad, offs)


def kernel(h, edge_index, e, W_enc, b_enc, W0, b0, W1, b1, W2, b2, W3, b3,
           W_ro, b_ro):
    src = edge_index[0]
    dst = edge_index[1]
    order = jnp.argsort(dst)
    src_s = src[order].astype(jnp.int32)
    dst_s = dst[order].astype(jnp.int32)
    pad = jnp.zeros((C,), jnp.int32)
    src_pad = jnp.concatenate([src_s, pad])
    dst_pad = jnp.concatenate([dst_s, pad])
    bounds = jnp.arange(0, NPAD + 1, NPT, dtype=jnp.int32)
    offs = jnp.searchsorted(dst_s, bounds).astype(jnp.int32)
    offs = jnp.concatenate([offs, jnp.full((7,), E, jnp.int32)])

    h = _matmul_bias(h, W_enc, b_enc)
    for W, b in ((W0, b0), (W1, b1), (W2, b2), (W3, b3)):
        s, mx, mn, sq, deg16 = _sc_aggregate(h, src_pad, dst_pad, offs)
        deg = deg16.reshape(NPAD, 16)[:N, 0:1]
        h = _layer_combine(h, s[:N], mx[:N], mn[:N], sq[:N], deg, W, b)
    return _matmul_bias(h, W_ro, b_ro)


# speedup vs baseline: 6.3179x; 1.3071x over previous
import functools

import jax
import jax.numpy as jnp
from jax import lax
from jax.experimental import pallas as pl
from jax.experimental.pallas import tpu as pltpu
from jax.experimental.pallas import tpu_sc as plsc

N = 10000
D = 128
E = 320000
AVG_D_LOG = 3.5

_ROW_BLK = 1000

NPT = 160
NW = 32
NWAVES = 2
NSLOT = NW * NWAVES
NPAD = NPT * NSLOT
GARB = NPT
ARWS = 168
C = 128
NEG = -3.0e38
POS = 3.0e38


def _mm_kernel(x_ref, w_ref, b_ref, o_ref, *, relu):
    acc = jnp.dot(x_ref[...], w_ref[...], preferred_element_type=jnp.float32)
    acc = acc + b_ref[...][None, :]
    if relu:
        acc = jnp.maximum(acc, 0.0)
    o_ref[...] = acc


def _matmul_bias(x, w, b, relu=False):
    n, k = x.shape
    m = w.shape[1]
    grid = (n // _ROW_BLK,)
    return pl.pallas_call(
        functools.partial(_mm_kernel, relu=relu),
        grid=grid,
        in_specs=[
            pl.BlockSpec((_ROW_BLK, k), lambda i: (i, 0)),
            pl.BlockSpec((k, m), lambda i: (0, 0)),
            pl.BlockSpec((m,), lambda i: (0,)),
        ],
        out_specs=pl.BlockSpec((_ROW_BLK, m), lambda i: (i, 0)),
        out_shape=jax.ShapeDtypeStruct((n, m), jnp.float32),
    )(x, w, b)


def _combine_kernel(h_ref, s_ref, mx_ref, mn_ref, sq_ref, deg_ref, w_ref,
                    b_ref, o_ref):
    deg = deg_ref[...]
    degc = jnp.maximum(deg, 1.0)
    invd = 1.0 / degc
    has = deg > 0.0
    mean = s_ref[...] * invd
    msq = sq_ref[...] * invd
    std = jnp.sqrt(jnp.maximum(msq - mean * mean, 0.0) + 1e-5)
    mx = jnp.where(has, mx_ref[...], 0.0)
    mn = jnp.where(has, mn_ref[...], 0.0)
    agg = jnp.concatenate([mean, mx, mn, std], axis=1)
    ld = jnp.log(deg + 1.0)
    amp = ld / AVG_D_LOG
    att = AVG_D_LOG / jnp.maximum(ld, 1e-5)
    w = w_ref[...]
    acc = jnp.dot(h_ref[...], w[0:D], preferred_element_type=jnp.float32)
    acc += jnp.dot(agg, w[D:D + 512], preferred_element_type=jnp.float32)
    acc += jnp.dot(agg * amp, w[D + 512:D + 1024],
                   preferred_element_type=jnp.float32)
    acc += jnp.dot(agg * att, w[D + 1024:D + 1536],
                   preferred_element_type=jnp.float32)
    acc += b_ref[...][None, :]
    o_ref[...] = jnp.maximum(acc, 0.0)


def _layer_combine(h, s, mx, mn, sq, deg, w, b):
    grid = (N // _ROW_BLK,)
    blk = lambda i: (i, 0)
    return pl.pallas_call(
        _combine_kernel,
        grid=grid,
        in_specs=[
            pl.BlockSpec((_ROW_BLK, D), blk),
            pl.BlockSpec((_ROW_BLK, D), blk),
            pl.BlockSpec((_ROW_BLK, D), blk),
            pl.BlockSpec((_ROW_BLK, D), blk),
            pl.BlockSpec((_ROW_BLK, D), blk),
            pl.BlockSpec((_ROW_BLK, 1), blk),
            pl.BlockSpec((13 * D, D), lambda i: (0, 0)),
            pl.BlockSpec((D,), lambda i: (0,)),
        ],
        out_specs=pl.BlockSpec((_ROW_BLK, D), blk),
        out_shape=jax.ShapeDtypeStruct((N, D), jnp.float32),
    )(h, s, mx, mn, sq, deg, w, b)


def _sc_agg_body(h_hbm, src_hbm, dst_hbm, offs_hbm,
                 o_s, o_mx, o_mn, o_sq, o_deg,
                 mx_v, mn_v, s_a, sq_a, deg_v,
                 rows0, rows1, src0, src1, dloc0, dloc1, offs_v,
                 sem_g0, sem_g1, sem_i0, sem_i1):
    cid = lax.axis_index("c")
    sid = lax.axis_index("s")
    wid = sid * 2 + cid

    iota = lax.iota(jnp.int32, 16)
    zeros16 = jnp.zeros((16,), jnp.float32)
    negv = jnp.full((16,), NEG, jnp.float32)
    posv = jnp.full((16,), POS, jnp.float32)
    ones16 = jnp.ones((16,), jnp.float32)
    cols = [iota + 16 * k for k in range(D // 16)]
    rows = (rows0, rows1)
    srcs = (src0, src1)
    dlocs = (dloc0, dloc1)
    sems_g = (sem_g0, sem_g1)
    sems_i = (sem_i0, sem_i1)

    pltpu.sync_copy(offs_hbm, offs_v)

    for wave in range(NWAVES):
        slot = wave * NW + wid
        base_node = slot * NPT

        def _init(i, _):
            for k in range(D // 16):
                mx_v[i, pl.ds(16 * k, 16)] = negv
                mn_v[i, pl.ds(16 * k, 16)] = posv
                s_a[i, pl.ds(16 * k, 16)] = zeros16
                sq_a[i, pl.ds(16 * k, 16)] = zeros16
            deg_v[pl.ds(16 * i, 16)] = zeros16
            return 0
        lax.fori_loop(0, ARWS, _init, 0)

        sltv = jnp.full((16,), slot, jnp.int32)
        start = lax.reduce_max(plsc.load_gather(offs_v, [sltv]), (0,))
        end = lax.reduce_max(plsc.load_gather(offs_v, [sltv + 1]), (0,))
        astart = start & ~7
        nchunks = (end - astart + (C - 1)) // C
        npairs = (nchunks + 1) >> 1

        def _cbase(c):
            return pl.multiple_of(astart + c * C, 8)

        def _grp(c, dl):
            cbase = _cbase(c)
            for g in range(C // 16):
                ids = jnp.full((16,), cbase + g * 16, jnp.int32) + iota
                valid = (ids >= start) & (ids < end)
                loc = dl[pl.ds(g * 16, 16)] - base_node
                dl[pl.ds(g * 16, 16)] = jnp.where(valid, loc, GARB)

        def _issue_idx(c, p):
            cbase = _cbase(c)
            pltpu.async_copy(src_hbm.at[pl.ds(cbase, C)], srcs[p], sems_i[p])
            pltpu.async_copy(dst_hbm.at[pl.ds(cbase, C)], dlocs[p], sems_i[p])

        def _drain_idx(p):
            pltpu.make_async_copy(src_hbm.at[pl.ds(0, C)], srcs[p],
                                  sems_i[p]).wait()
            pltpu.make_async_copy(dst_hbm.at[pl.ds(0, C)], dlocs[p],
                                  sems_i[p]).wait()

        def _issue_gather(p):
            pltpu.async_copy(h_hbm.at[srcs[p]], rows[p], sems_g[p])

        def _drain_gather(p):
            pltpu.make_async_copy(h_hbm.at[pl.ds(0, C)], rows[p],
                                  sems_g[p]).wait()

        def _edges(p):
            rp = rows[p]
            dp = dlocs[p]

            def _edge(j, _):
                bvec = plsc.load_gather(dp, [jnp.full((16,), j, jnp.int32)])
                dgi = bvec * 16 + iota
                ms = [rp[j, pl.ds(16 * k, 16)] for k in range(D // 16)]
                cmx = [plsc.load_gather(mx_v, [bvec, cols[k]])
                       for k in range(D // 16)]
                cmn = [plsc.load_gather(mn_v, [bvec, cols[k]])
                       for k in range(D // 16)]
                for k in range(D // 16):
                    plsc.store_scatter(mx_v, [bvec, cols[k]],
                                       jnp.maximum(cmx[k], ms[k]))
                    plsc.store_scatter(mn_v, [bvec, cols[k]],
                                       jnp.minimum(cmn[k], ms[k]))
                for k in range(D // 16):
                    plsc.addupdate_scatter(s_a, [bvec, cols[k]], ms[k])
                    plsc.addupdate_scatter(sq_a, [bvec, cols[k]],
                                           ms[k] * ms[k])
                plsc.addupdate_scatter(deg_v, [dgi], ones16)
                return 0
            lax.fori_loop(0, C, _edge, 0)

        a0 = pl.multiple_of(astart, 8)
        pltpu.sync_copy(src_hbm.at[pl.ds(a0, C)], src0)
        pltpu.sync_copy(dst_hbm.at[pl.ds(a0, C)], dloc0)
        _grp(0, dloc0)
        _issue_gather(0)
        _issue_idx(1, 1)

        def _chunk(c, p):
            q = 1 - p
            _drain_idx(q)
            _grp(c + 1, dlocs[q])
            _issue_gather(q)
            _drain_gather(p)
            _edges(p)
            _issue_idx(c + 2, p)

        def _pair(i, _):
            _chunk(2 * i, 0)
            _chunk(2 * i + 1, 1)
            return 0
        lax.fori_loop(0, npairs, _pair, 0)

        _drain_gather(0)
        _drain_idx(1)

        pltpu.sync_copy(mx_v.at[pl.ds(0, NPT)],
                        o_mx.at[pl.ds(base_node, NPT)])
        pltpu.sync_copy(mn_v.at[pl.ds(0, NPT)],
                        o_mn.at[pl.ds(base_node, NPT)])
        pltpu.sync_copy(s_a.at[pl.ds(0, NPT)],
                        o_s.at[pl.ds(base_node, NPT)])
        pltpu.sync_copy(sq_a.at[pl.ds(0, NPT)],
                        o_sq.at[pl.ds(base_node, NPT)])
        pltpu.sync_copy(deg_v.at[pl.ds(0, NPT * 16)],
                        o_deg.at[pl.ds(pl.multiple_of(base_node * 16, 8),
                                       NPT * 16)])


def _sc_aggregate(h, src_pad, dst_pad, offs):
    mesh = plsc.VectorSubcoreMesh(core_axis_name="c", subcore_axis_name="s")
    f32 = jnp.float32
    out_type = [
        jax.ShapeDtypeStruct((NPAD, D), f32),
        jax.ShapeDtypeStruct((NPAD, D), f32),
        jax.ShapeDtypeStruct((NPAD, D), f32),
        jax.ShapeDtypeStruct((NPAD, D), f32),
        jax.ShapeDtypeStruct((NPAD * 16,), f32),
    ]
    scratch = [
        pltpu.VMEM((ARWS, D), f32),
        pltpu.VMEM((ARWS, D), f32),
        pltpu.VMEM((ARWS, D), f32),
        pltpu.VMEM((ARWS, D), f32),
        pltpu.VMEM((ARWS * 16,), f32),
        pltpu.VMEM((C, D), f32),
        pltpu.VMEM((C, D), f32),
        pltpu.VMEM((C,), jnp.int32),
        pltpu.VMEM((C,), jnp.int32),
        pltpu.VMEM((C,), jnp.int32),
        pltpu.VMEM((C,), jnp.int32),
        pltpu.VMEM((NSLOT + 8,), jnp.int32),
        pltpu.SemaphoreType.DMA,
        pltpu.SemaphoreType.DMA,
        pltpu.SemaphoreType.DMA,
        pltpu.SemaphoreType.DMA,
    ]
    kern = pl.kernel(_sc_agg_body, out_type=out_type, mesh=mesh,
                     scratch_types=scratch,
                     compiler_params=pltpu.CompilerParams(
                         needs_layout_passes=False))
    return kern(h, src_pad, dst_pad, offs)


def kernel(h, edge_index, e, W_enc, b_enc, W0, b0, W1, b1, W2, b2, W3, b3,
           W_ro, b_ro):
    src = edge_index[0]
    dst = edge_index[1]
    order = jnp.argsort(dst)
    src_s = src[order].astype(jnp.int32)
    dst_s = dst[order].astype(jnp.int32)
    pad = jnp.zeros((4 * C,), jnp.int32)
    src_pad = jnp.concatenate([src_s, pad])
    dst_pad = jnp.concatenate([dst_s, pad])
    bounds = jnp.arange(0, NPAD + 1, NPT, dtype=jnp.int32)
    offs = jnp.searchsorted(dst_s, bounds).astype(jnp.int32)
    offs = jnp.concatenate([offs, jnp.full((7,), E, jnp.int32)])

    h = _matmul_bias(h, W_enc, b_enc)
    for W, b in ((W0, b0), (W1, b1), (W2, b2), (W3, b3)):
        s, mx, mn, sq, deg16 = _sc_aggregate(h, src_pad, dst_pad, offs)
        deg = deg16.reshape(NPAD, 16)[:N, 0:1]
        h = _layer_combine(h, s[:N], mx[:N], mn[:N], sq[:N], deg, W, b)
    return _matmul_bias(h, W_ro, b_ro)


# speedup vs baseline: 6.3727x; 1.0087x over previous
import functools

import jax
import jax.numpy as jnp
from jax import lax
from jax.experimental import pallas as pl
from jax.experimental.pallas import tpu as pltpu
from jax.experimental.pallas import tpu_sc as plsc

N = 10000
D = 128
E = 320000
AVG_D_LOG = 3.5

_ROW_BLK = 1000

NPT = 160
NW = 32
NWAVES = 2
NSLOT = NW * NWAVES
NPAD = NPT * NSLOT
GARB = NPT
ARWS = 168
C = 128
NEG = -3.0e38
POS = 3.0e38


def _mm_kernel(x_ref, w_ref, b_ref, o_ref, *, relu):
    acc = jnp.dot(x_ref[...], w_ref[...], preferred_element_type=jnp.float32)
    acc = acc + b_ref[...][None, :]
    if relu:
        acc = jnp.maximum(acc, 0.0)
    o_ref[...] = acc


def _matmul_bias(x, w, b, relu=False):
    n, k = x.shape
    m = w.shape[1]
    grid = (n // _ROW_BLK,)
    return pl.pallas_call(
        functools.partial(_mm_kernel, relu=relu),
        grid=grid,
        in_specs=[
            pl.BlockSpec((_ROW_BLK, k), lambda i: (i, 0)),
            pl.BlockSpec((k, m), lambda i: (0, 0)),
            pl.BlockSpec((m,), lambda i: (0,)),
        ],
        out_specs=pl.BlockSpec((_ROW_BLK, m), lambda i: (i, 0)),
        out_shape=jax.ShapeDtypeStruct((n, m), jnp.float32),
    )(x, w, b)


def _combine_kernel(h_ref, s_ref, mx_ref, mn_ref, sq_ref, deg_ref, w_ref,
                    b_ref, o_ref):
    deg = deg_ref[...]
    degc = jnp.maximum(deg, 1.0)
    invd = 1.0 / degc
    has = deg > 0.0
    mean = s_ref[...] * invd
    msq = sq_ref[...] * invd
    std = jnp.sqrt(jnp.maximum(msq - mean * mean, 0.0) + 1e-5)
    mx = jnp.where(has, mx_ref[...], 0.0)
    mn = jnp.where(has, mn_ref[...], 0.0)
    agg = jnp.concatenate([mean, mx, mn, std], axis=1)
    ld = jnp.log(deg + 1.0)
    amp = ld / AVG_D_LOG
    att = AVG_D_LOG / jnp.maximum(ld, 1e-5)
    w = w_ref[...]
    acc = jnp.dot(h_ref[...], w[0:D], preferred_element_type=jnp.float32)
    acc += jnp.dot(agg, w[D:D + 512], preferred_element_type=jnp.float32)
    acc += jnp.dot(agg * amp, w[D + 512:D + 1024],
                   preferred_element_type=jnp.float32)
    acc += jnp.dot(agg * att, w[D + 1024:D + 1536],
                   preferred_element_type=jnp.float32)
    acc += b_ref[...][None, :]
    o_ref[...] = jnp.maximum(acc, 0.0)


def _layer_combine(h, s, mx, mn, sq, deg, w, b):
    grid = (N // _ROW_BLK,)
    blk = lambda i: (i, 0)
    return pl.pallas_call(
        _combine_kernel,
        grid=grid,
        in_specs=[
            pl.BlockSpec((_ROW_BLK, D), blk),
            pl.BlockSpec((_ROW_BLK, D), blk),
            pl.BlockSpec((_ROW_BLK, D), blk),
            pl.BlockSpec((_ROW_BLK, D), blk),
            pl.BlockSpec((_ROW_BLK, D), blk),
            pl.BlockSpec((_ROW_BLK, 1), blk),
            pl.BlockSpec((13 * D, D), lambda i: (0, 0)),
            pl.BlockSpec((D,), lambda i: (0,)),
        ],
        out_specs=pl.BlockSpec((_ROW_BLK, D), blk),
        out_shape=jax.ShapeDtypeStruct((N, D), jnp.float32),
    )(h, s, mx, mn, sq, deg, w, b)


def _sc_agg_body(h_hbm, src_hbm, dst_hbm, offs_hbm,
                 o_s, o_mx, o_mn, o_sq, o_deg,
                 mx_v, mn_v, s_a, sq_a, deg_v,
                 rows0, rows1, src0, src1, dloc0, dloc1, offs_v,
                 sem_g0, sem_g1, sem_i0, sem_i1):
    cid = lax.axis_index("c")
    sid = lax.axis_index("s")
    wid = sid * 2 + cid

    iota = lax.iota(jnp.int32, 16)
    zeros16 = jnp.zeros((16,), jnp.float32)
    negv = jnp.full((16,), NEG, jnp.float32)
    posv = jnp.full((16,), POS, jnp.float32)
    ones16 = jnp.ones((16,), jnp.float32)
    cols = [iota + 16 * k for k in range(D // 16)]
    rows = (rows0, rows1)
    srcs = (src0, src1)
    dlocs = (dloc0, dloc1)
    sems_g = (sem_g0, sem_g1)
    sems_i = (sem_i0, sem_i1)

    pltpu.sync_copy(offs_hbm, offs_v)

    for wave in range(NWAVES):
        slot = wave * NW + wid
        base_node = slot * NPT

        def _init(i, _):
            for k in range(D // 16):
                mx_v[i, pl.ds(16 * k, 16)] = negv
                mn_v[i, pl.ds(16 * k, 16)] = posv
                s_a[i, pl.ds(16 * k, 16)] = zeros16
                sq_a[i, pl.ds(16 * k, 16)] = zeros16
            deg_v[pl.ds(16 * i, 16)] = zeros16
            return 0
        lax.fori_loop(0, ARWS, _init, 0)

        sltv = jnp.full((16,), slot, jnp.int32)
        start = lax.reduce_max(plsc.load_gather(offs_v, [sltv]), (0,))
        end = lax.reduce_max(plsc.load_gather(offs_v, [sltv + 1]), (0,))
        astart = start & ~7
        nchunks = (end - astart + (C - 1)) // C
        npairs = (nchunks + 1) >> 1

        def _cbase(c):
            return pl.multiple_of(astart + c * C, 8)

        def _grp(c, dl):
            cbase = _cbase(c)
            for g in range(C // 16):
                ids = jnp.full((16,), cbase + g * 16, jnp.int32) + iota
                valid = (ids >= start) & (ids < end)
                loc = dl[pl.ds(g * 16, 16)] - base_node
                dl[pl.ds(g * 16, 16)] = jnp.where(valid, loc, GARB)

        def _issue_idx(c, p):
            cbase = _cbase(c)
            pltpu.async_copy(src_hbm.at[pl.ds(cbase, C)], srcs[p], sems_i[p])
            pltpu.async_copy(dst_hbm.at[pl.ds(cbase, C)], dlocs[p], sems_i[p])

        def _drain_idx(p):
            pltpu.make_async_copy(src_hbm.at[pl.ds(0, C)], srcs[p],
                                  sems_i[p]).wait()
            pltpu.make_async_copy(dst_hbm.at[pl.ds(0, C)], dlocs[p],
                                  sems_i[p]).wait()

        def _issue_gather(p):
            pltpu.async_copy(h_hbm.at[srcs[p]], rows[p], sems_g[p])

        def _drain_gather(p):
            pltpu.make_async_copy(h_hbm.at[pl.ds(0, C)], rows[p],
                                  sems_g[p]).wait()

        def _edges(p):
            rp = rows[p]
            dp = dlocs[p]

            def _edge(jj, _):
                jb = jj * 4
                for u in range(4):
                    j = jb + u
                    bvec = plsc.load_gather(
                        dp, [jnp.full((16,), j, jnp.int32)])
                    dgi = bvec * 16 + iota
                    ms = [rp[j, pl.ds(16 * k, 16)] for k in range(D // 16)]
                    cmx = [plsc.load_gather(mx_v, [bvec, cols[k]])
                           for k in range(D // 16)]
                    cmn = [plsc.load_gather(mn_v, [bvec, cols[k]])
                           for k in range(D // 16)]
                    for k in range(D // 16):
                        plsc.store_scatter(mx_v, [bvec, cols[k]],
                                           jnp.maximum(cmx[k], ms[k]))
                        plsc.store_scatter(mn_v, [bvec, cols[k]],
                                           jnp.minimum(cmn[k], ms[k]))
                    for k in range(D // 16):
                        plsc.addupdate_scatter(s_a, [bvec, cols[k]], ms[k])
                        plsc.addupdate_scatter(sq_a, [bvec, cols[k]],
                                               ms[k] * ms[k])
                    plsc.addupdate_scatter(deg_v, [dgi], ones16)
                return 0
            lax.fori_loop(0, C // 4, _edge, 0)

        a0 = pl.multiple_of(astart, 8)
        pltpu.sync_copy(src_hbm.at[pl.ds(a0, C)], src0)
        pltpu.sync_copy(dst_hbm.at[pl.ds(a0, C)], dloc0)
        _grp(0, dloc0)
        _issue_gather(0)
        _issue_idx(1, 1)

        def _chunk(c, p):
            q = 1 - p
            _drain_idx(q)
            _grp(c + 1, dlocs[q])
            _issue_gather(q)
            _drain_gather(p)
            _edges(p)
            _issue_idx(c + 2, p)

        def _pair(i, _):
            _chunk(2 * i, 0)
            _chunk(2 * i + 1, 1)
            return 0
        lax.fori_loop(0, npairs, _pair, 0)

        _drain_gather(0)
        _drain_idx(1)

        pltpu.sync_copy(mx_v.at[pl.ds(0, NPT)],
                        o_mx.at[pl.ds(base_node, NPT)])
        pltpu.sync_copy(mn_v.at[pl.ds(0, NPT)],
                        o_mn.at[pl.ds(base_node, NPT)])
        pltpu.sync_copy(s_a.at[pl.ds(0, NPT)],
                        o_s.at[pl.ds(base_node, NPT)])
        pltpu.sync_copy(sq_a.at[pl.ds(0, NPT)],
                        o_sq.at[pl.ds(base_node, NPT)])
        pltpu.sync_copy(deg_v.at[pl.ds(0, NPT * 16)],
                        o_deg.at[pl.ds(pl.multiple_of(base_node * 16, 8),
                                       NPT * 16)])


def _sc_aggregate(h, src_pad, dst_pad, offs):
    mesh = plsc.VectorSubcoreMesh(core_axis_name="c", subcore_axis_name="s")
    f32 = jnp.float32
    out_type = [
        jax.ShapeDtypeStruct((NPAD, D), f32),
        jax.ShapeDtypeStruct((NPAD, D), f32),
        jax.ShapeDtypeStruct((NPAD, D), f32),
        jax.ShapeDtypeStruct((NPAD, D), f32),
        jax.ShapeDtypeStruct((NPAD * 16,), f32),
    ]
    scratch = [
        pltpu.VMEM((ARWS, D), f32),
        pltpu.VMEM((ARWS, D), f32),
        pltpu.VMEM((ARWS, D), f32),
        pltpu.VMEM((ARWS, D), f32),
        pltpu.VMEM((ARWS * 16,), f32),
        pltpu.VMEM((C, D), f32),
        pltpu.VMEM((C, D), f32),
        pltpu.VMEM((C,), jnp.int32),
        pltpu.VMEM((C,), jnp.int32),
        pltpu.VMEM((C,), jnp.int32),
        pltpu.VMEM((C,), jnp.int32),
        pltpu.VMEM((NSLOT + 8,), jnp.int32),
        pltpu.SemaphoreType.DMA,
        pltpu.SemaphoreType.DMA,
        pltpu.SemaphoreType.DMA,
        pltpu.SemaphoreType.DMA,
    ]
    kern = pl.kernel(_sc_agg_body, out_type=out_type, mesh=mesh,
                     scratch_types=scratch,
                     compiler_params=pltpu.CompilerParams(
                         needs_layout_passes=False))
    return kern(h, src_pad, dst_pad, offs)


def kernel(h, edge_index, e, W_enc, b_enc, W0, b0, W1, b1, W2, b2, W3, b3,
           W_ro, b_ro):
    src = edge_index[0]
    dst = edge_index[1]
    order = jnp.argsort(dst)
    src_s = src[order].astype(jnp.int32)
    dst_s = dst[order].astype(jnp.int32)
    pad = jnp.zeros((4 * C,), jnp.int32)
    src_pad = jnp.concatenate([src_s, pad])
    dst_pad = jnp.concatenate([dst_s, pad])
    bounds = jnp.arange(0, NPAD + 1, NPT, dtype=jnp.int32)
    offs = jnp.searchsorted(dst_s, bounds).astype(jnp.int32)
    offs = jnp.concatenate([offs, jnp.full((7,), E, jnp.int32)])

    h = _matmul_bias(h, W_enc, b_enc)
    for W, b in ((W0, b0), (W1, b1), (W2, b2), (W3, b3)):
        s, mx, mn, sq, deg16 = _sc_aggregate(h, src_pad, dst_pad, offs)
        deg = deg16.reshape(NPAD, 16)[:N, 0:1]
        h = _layer_combine(h, s[:N], mx[:N], mn[:N], sq[:N], deg, W, b)
    return _matmul_bias(h, W_ro, b_ro)


# speedup vs baseline: 8.5733x; 1.3453x over previous
import functools

import jax
import jax.numpy as jnp
from jax import lax
from jax.experimental import pallas as pl
from jax.experimental.pallas import tpu as pltpu
from jax.experimental.pallas import tpu_sc as plsc

N = 10000
D = 128
E = 320000
AVG_D_LOG = 3.5

_ROW_BLK = 1000

NPT = 160
NW = 32
NWAVES = 2
NSLOT = NW * NWAVES
NPAD = NPT * NSLOT
GARB = NPT
ARWS = 168
C = 128
NEG = -3.0e38
POS = 3.0e38


def _mm_kernel(x_ref, w_ref, b_ref, o_ref, *, relu):
    acc = jnp.dot(x_ref[...], w_ref[...], preferred_element_type=jnp.float32)
    acc = acc + b_ref[...][None, :]
    if relu:
        acc = jnp.maximum(acc, 0.0)
    o_ref[...] = acc


def _matmul_bias(x, w, b, relu=False):
    n, k = x.shape
    m = w.shape[1]
    grid = (n // _ROW_BLK,)
    return pl.pallas_call(
        functools.partial(_mm_kernel, relu=relu),
        grid=grid,
        in_specs=[
            pl.BlockSpec((_ROW_BLK, k), lambda i: (i, 0)),
            pl.BlockSpec((k, m), lambda i: (0, 0)),
            pl.BlockSpec((m,), lambda i: (0,)),
        ],
        out_specs=pl.BlockSpec((_ROW_BLK, m), lambda i: (i, 0)),
        out_shape=jax.ShapeDtypeStruct((n, m), jnp.float32),
    )(x, w, b)


def _combine_kernel(h_ref, s_ref, mx_ref, mn_ref, sq_ref, deg_ref, w_ref,
                    b_ref, o_ref):
    deg = deg_ref[...]
    degc = jnp.maximum(deg, 1.0)
    invd = 1.0 / degc
    has = deg > 0.0
    mean = s_ref[...] * invd
    msq = sq_ref[...] * invd
    std = jnp.sqrt(jnp.maximum(msq - mean * mean, 0.0) + 1e-5)
    mx = jnp.where(has, mx_ref[...], 0.0)
    mn = jnp.where(has, mn_ref[...], 0.0)
    agg = jnp.concatenate([mean, mx, mn, std], axis=1)
    ld = jnp.log(deg + 1.0)
    amp = ld / AVG_D_LOG
    att = AVG_D_LOG / jnp.maximum(ld, 1e-5)
    w = w_ref[...]
    acc = jnp.dot(h_ref[...], w[0:D], preferred_element_type=jnp.float32)
    acc += jnp.dot(agg, w[D:D + 512], preferred_element_type=jnp.float32)
    acc += jnp.dot(agg * amp, w[D + 512:D + 1024],
                   preferred_element_type=jnp.float32)
    acc += jnp.dot(agg * att, w[D + 1024:D + 1536],
                   preferred_element_type=jnp.float32)
    acc += b_ref[...][None, :]
    o_ref[...] = jnp.maximum(acc, 0.0)


def _layer_combine(h, s, mx, mn, sq, deg, w, b):
    grid = (N // _ROW_BLK,)
    blk = lambda i: (i, 0)
    return pl.pallas_call(
        _combine_kernel,
        grid=grid,
        in_specs=[
            pl.BlockSpec((_ROW_BLK, D), blk),
            pl.BlockSpec((_ROW_BLK, D), blk),
            pl.BlockSpec((_ROW_BLK, D), blk),
            pl.BlockSpec((_ROW_BLK, D), blk),
            pl.BlockSpec((_ROW_BLK, D), blk),
            pl.BlockSpec((_ROW_BLK, 1), blk),
            pl.BlockSpec((13 * D, D), lambda i: (0, 0)),
            pl.BlockSpec((D,), lambda i: (0,)),
        ],
        out_specs=pl.BlockSpec((_ROW_BLK, D), blk),
        out_shape=jax.ShapeDtypeStruct((N, D), jnp.float32),
    )(h, s, mx, mn, sq, deg, w, b)


def _sc_agg_body(h_hbm, src_hbm, dst_hbm, offs_hbm,
                 o_s, o_mx, o_mn, o_sq, o_deg,
                 mx_v, mn_v, s_a, sq_a, deg_v,
                 rows0, rows1, src0, src1, dloc0, dloc1, offs_v,
                 sem_g0, sem_g1, sem_i0, sem_i1):
    cid = lax.axis_index("c")
    sid = lax.axis_index("s")
    wid = sid * 2 + cid

    iota = lax.iota(jnp.int32, 16)
    zeros16 = jnp.zeros((16,), jnp.float32)
    negv = jnp.full((16,), NEG, jnp.float32)
    posv = jnp.full((16,), POS, jnp.float32)
    ones16 = jnp.ones((16,), jnp.float32)
    cols = [iota + 16 * k for k in range(D // 16)]
    rows = (rows0, rows1)
    srcs = (src0, src1)
    dlocs = (dloc0, dloc1)
    sems_g = (sem_g0, sem_g1)
    sems_i = (sem_i0, sem_i1)

    pltpu.sync_copy(offs_hbm, offs_v)

    for wave in range(NWAVES):
        slot = wave * NW + wid
        base_node = slot * NPT

        def _init(i, _):
            for k in range(D // 16):
                mx_v[i, pl.ds(16 * k, 16)] = negv
                mn_v[i, pl.ds(16 * k, 16)] = posv
                s_a[i, pl.ds(16 * k, 16)] = zeros16
                sq_a[i, pl.ds(16 * k, 16)] = zeros16
            deg_v[pl.ds(16 * i, 16)] = zeros16
            return 0
        lax.fori_loop(0, ARWS, _init, 0)

        sltv = jnp.full((16,), slot, jnp.int32)
        start = lax.reduce_max(plsc.load_gather(offs_v, [sltv]), (0,))
        end = lax.reduce_max(plsc.load_gather(offs_v, [sltv + 1]), (0,))
        astart = start & ~7
        nchunks = (end - astart + (C - 1)) // C
        npairs = (nchunks + 1) >> 1

        def _cbase(c):
            return pl.multiple_of(astart + c * C, 8)

        def _grp(c, dl):
            cbase = _cbase(c)
            for g in range(C // 16):
                ids = jnp.full((16,), cbase + g * 16, jnp.int32) + iota
                valid = (ids >= start) & (ids < end)
                loc = dl[pl.ds(g * 16, 16)] - base_node
                dl[pl.ds(g * 16, 16)] = jnp.where(valid, loc, GARB)

        def _issue_idx(c, p):
            cbase = _cbase(c)
            pltpu.async_copy(src_hbm.at[pl.ds(cbase, C)], srcs[p], sems_i[p])
            pltpu.async_copy(dst_hbm.at[pl.ds(cbase, C)], dlocs[p], sems_i[p])

        def _drain_idx(p):
            pltpu.make_async_copy(src_hbm.at[pl.ds(0, C)], srcs[p],
                                  sems_i[p]).wait()
            pltpu.make_async_copy(dst_hbm.at[pl.ds(0, C)], dlocs[p],
                                  sems_i[p]).wait()

        def _issue_gather(p):
            pltpu.async_copy(h_hbm.at[srcs[p]], rows[p], sems_g[p])

        def _drain_gather(p):
            pltpu.make_async_copy(h_hbm.at[pl.ds(0, C)], rows[p],
                                  sems_g[p]).wait()

        def _edges(p):
            rp = rows[p]
            dp = dlocs[p]

            fours16 = jnp.full((16,), 4.0, jnp.float32)

            def _edge(jj, _):
                jb = jj * 4
                bv = [plsc.load_gather(dp, [jnp.full((16,), jb + u,
                                                     jnp.int32)])
                      for u in range(4)]
                mm = [[rp[jb + u, pl.ds(16 * k, 16)]
                       for k in range(D // 16)] for u in range(4)]
                same = ((bv[0] == bv[1]) & (bv[1] == bv[2])
                        & (bv[2] == bv[3]))
                allsame = jnp.all(same)

                @pl.when(allsame)
                def _fast():
                    b = bv[0]
                    cmx = [plsc.load_gather(mx_v, [b, cols[k]])
                           for k in range(D // 16)]
                    cmn = [plsc.load_gather(mn_v, [b, cols[k]])
                           for k in range(D // 16)]
                    for k in range(D // 16):
                        m0, m1, m2, m3 = (mm[0][k], mm[1][k], mm[2][k],
                                          mm[3][k])
                        gmx = jnp.maximum(jnp.maximum(m0, m1),
                                          jnp.maximum(m2, m3))
                        gmn = jnp.minimum(jnp.minimum(m0, m1),
                                          jnp.minimum(m2, m3))
                        gs = (m0 + m1) + (m2 + m3)
                        gq = (m0 * m0 + m1 * m1) + (m2 * m2 + m3 * m3)
                        plsc.store_scatter(mx_v, [b, cols[k]],
                                           jnp.maximum(cmx[k], gmx))
                        plsc.store_scatter(mn_v, [b, cols[k]],
                                           jnp.minimum(cmn[k], gmn))
                        plsc.addupdate_scatter(s_a, [b, cols[k]], gs)
                        plsc.addupdate_scatter(sq_a, [b, cols[k]], gq)
                    plsc.addupdate_scatter(deg_v, [b * 16 + iota], fours16)

                @pl.when(jnp.logical_not(allsame))
                def _slow():
                    for u in range(4):
                        bvec = bv[u]
                        dgi = bvec * 16 + iota
                        ms = mm[u]
                        cmx = [plsc.load_gather(mx_v, [bvec, cols[k]])
                               for k in range(D // 16)]
                        cmn = [plsc.load_gather(mn_v, [bvec, cols[k]])
                               for k in range(D // 16)]
                        for k in range(D // 16):
                            plsc.store_scatter(mx_v, [bvec, cols[k]],
                                               jnp.maximum(cmx[k], ms[k]))
                            plsc.store_scatter(mn_v, [bvec, cols[k]],
                                               jnp.minimum(cmn[k], ms[k]))
                        for k in range(D // 16):
                            plsc.addupdate_scatter(s_a, [bvec, cols[k]],
                                                   ms[k])
                            plsc.addupdate_scatter(sq_a, [bvec, cols[k]],
                                                   ms[k] * ms[k])
                        plsc.addupdate_scatter(deg_v, [dgi], ones16)
                return 0
            lax.fori_loop(0, C // 4, _edge, 0)

        a0 = pl.multiple_of(astart, 8)
        pltpu.sync_copy(src_hbm.at[pl.ds(a0, C)], src0)
        pltpu.sync_copy(dst_hbm.at[pl.ds(a0, C)], dloc0)
        _grp(0, dloc0)
        _issue_gather(0)
        _issue_idx(1, 1)

        def _chunk(c, p):
            q = 1 - p
            _drain_idx(q)
            _grp(c + 1, dlocs[q])
            _issue_gather(q)
            _drain_gather(p)
            _edges(p)
            _issue_idx(c + 2, p)

        def _pair(i, _):
            _chunk(2 * i, 0)
            _chunk(2 * i + 1, 1)
            return 0
        lax.fori_loop(0, npairs, _pair, 0)

        _drain_gather(0)
        _drain_idx(1)

        pltpu.sync_copy(mx_v.at[pl.ds(0, NPT)],
                        o_mx.at[pl.ds(base_node, NPT)])
        pltpu.sync_copy(mn_v.at[pl.ds(0, NPT)],
                        o_mn.at[pl.ds(base_node, NPT)])
        pltpu.sync_copy(s_a.at[pl.ds(0, NPT)],
                        o_s.at[pl.ds(base_node, NPT)])
        pltpu.sync_copy(sq_a.at[pl.ds(0, NPT)],
                        o_sq.at[pl.ds(base_node, NPT)])
        pltpu.sync_copy(deg_v.at[pl.ds(0, NPT * 16)],
                        o_deg.at[pl.ds(pl.multiple_of(base_node * 16, 8),
                                       NPT * 16)])


def _sc_aggregate(h, src_pad, dst_pad, offs):
    mesh = plsc.VectorSubcoreMesh(core_axis_name="c", subcore_axis_name="s")
    f32 = jnp.float32
    out_type = [
        jax.ShapeDtypeStruct((NPAD, D), f32),
        jax.ShapeDtypeStruct((NPAD, D), f32),
        jax.ShapeDtypeStruct((NPAD, D), f32),
        jax.ShapeDtypeStruct((NPAD, D), f32),
        jax.ShapeDtypeStruct((NPAD * 16,), f32),
    ]
    scratch = [
        pltpu.VMEM((ARWS, D), f32),
        pltpu.VMEM((ARWS, D), f32),
        pltpu.VMEM((ARWS, D), f32),
        pltpu.VMEM((ARWS, D), f32),
        pltpu.VMEM((ARWS * 16,), f32),
        pltpu.VMEM((C, D), f32),
        pltpu.VMEM((C, D), f32),
        pltpu.VMEM((C,), jnp.int32),
        pltpu.VMEM((C,), jnp.int32),
        pltpu.VMEM((C,), jnp.int32),
        pltpu.VMEM((C,), jnp.int32),
        pltpu.VMEM((NSLOT + 8,), jnp.int32),
        pltpu.SemaphoreType.DMA,
        pltpu.SemaphoreType.DMA,
        pltpu.SemaphoreType.DMA,
        pltpu.SemaphoreType.DMA,
    ]
    kern = pl.kernel(_sc_agg_body, out_type=out_type, mesh=mesh,
                     scratch_types=scratch,
                     compiler_params=pltpu.CompilerParams(
                         needs_layout_passes=False))
    return kern(h, src_pad, dst_pad, offs)


def kernel(h, edge_index, e, W_enc, b_enc, W0, b0, W1, b1, W2, b2, W3, b3,
           W_ro, b_ro):
    src = edge_index[0]
    dst = edge_index[1]
    order = jnp.argsort(dst)
    src_s = src[order].astype(jnp.int32)
    dst_s = dst[order].astype(jnp.int32)
    pad = jnp.zeros((4 * C,), jnp.int32)
    src_pad = jnp.concatenate([src_s, pad])
    dst_pad = jnp.concatenate([dst_s, pad])
    bounds = jnp.arange(0, NPAD + 1, NPT, dtype=jnp.int32)
    offs = jnp.searchsorted(dst_s, bounds).astype(jnp.int32)
    offs = jnp.concatenate([offs, jnp.full((7,), E, jnp.int32)])

    h = _matmul_bias(h, W_enc, b_enc)
    for W, b in ((W0, b0), (W1, b1), (W2, b2), (W3, b3)):
        s, mx, mn, sq, deg16 = _sc_aggregate(h, src_pad, dst_pad, offs)
        deg = deg16.reshape(NPAD, 16)[:N, 0:1]
        h = _layer_combine(h, s[:N], mx[:N], mn[:N], sq[:N], deg, W, b)
    return _matmul_bias(h, W_ro, b_ro)


# speedup vs baseline: 9.2828x; 1.0828x over previous
import functools

import jax
import jax.numpy as jnp
from jax import lax
from jax.experimental import pallas as pl
from jax.experimental.pallas import tpu as pltpu
from jax.experimental.pallas import tpu_sc as plsc

N = 10000
D = 128
E = 320000
AVG_D_LOG = 3.5

_ROW_BLK = 1000

NPT = 160
NW = 32
NWAVES = 2
NSLOT = NW * NWAVES
NPAD = NPT * NSLOT
GARB = NPT
ARWS = 168
C = 128
NEG = -3.0e38
POS = 3.0e38


def _mm_kernel(x_ref, w_ref, b_ref, o_ref, *, relu):
    acc = jnp.dot(x_ref[...], w_ref[...], preferred_element_type=jnp.float32)
    acc = acc + b_ref[...][None, :]
    if relu:
        acc = jnp.maximum(acc, 0.0)
    o_ref[...] = acc


def _matmul_bias(x, w, b, relu=False):
    n, k = x.shape
    m = w.shape[1]
    grid = (n // _ROW_BLK,)
    return pl.pallas_call(
        functools.partial(_mm_kernel, relu=relu),
        grid=grid,
        in_specs=[
            pl.BlockSpec((_ROW_BLK, k), lambda i: (i, 0)),
            pl.BlockSpec((k, m), lambda i: (0, 0)),
            pl.BlockSpec((m,), lambda i: (0,)),
        ],
        out_specs=pl.BlockSpec((_ROW_BLK, m), lambda i: (i, 0)),
        out_shape=jax.ShapeDtypeStruct((n, m), jnp.float32),
    )(x, w, b)


def _combine_kernel(h_ref, s_ref, mx_ref, mn_ref, sq_ref, deg_ref, w_ref,
                    b_ref, o_ref):
    deg = deg_ref[...][:, 0:1]
    degc = jnp.maximum(deg, 1.0)
    invd = 1.0 / degc
    has = deg > 0.0
    mean = s_ref[...] * invd
    msq = sq_ref[...] * invd
    std = jnp.sqrt(jnp.maximum(msq - mean * mean, 0.0) + 1e-5)
    mx = jnp.where(has, mx_ref[...], 0.0)
    mn = jnp.where(has, mn_ref[...], 0.0)
    agg = jnp.concatenate([mean, mx, mn, std], axis=1)
    ld = jnp.log(deg + 1.0)
    amp = ld / AVG_D_LOG
    att = AVG_D_LOG / jnp.maximum(ld, 1e-5)
    w = w_ref[...]
    acc = jnp.dot(h_ref[...], w[0:D], preferred_element_type=jnp.float32)
    acc += jnp.dot(agg, w[D:D + 512], preferred_element_type=jnp.float32)
    acc += jnp.dot(agg * amp, w[D + 512:D + 1024],
                   preferred_element_type=jnp.float32)
    acc += jnp.dot(agg * att, w[D + 1024:D + 1536],
                   preferred_element_type=jnp.float32)
    acc += b_ref[...][None, :]
    o_ref[...] = jnp.maximum(acc, 0.0)


def _layer_combine(h, s, mx, mn, sq, deg, w, b):
    grid = (N // _ROW_BLK,)
    blk = lambda i: (i, 0)
    return pl.pallas_call(
        _combine_kernel,
        grid=grid,
        in_specs=[
            pl.BlockSpec((_ROW_BLK, D), blk),
            pl.BlockSpec((_ROW_BLK, D), blk),
            pl.BlockSpec((_ROW_BLK, D), blk),
            pl.BlockSpec((_ROW_BLK, D), blk),
            pl.BlockSpec((_ROW_BLK, D), blk),
            pl.BlockSpec((_ROW_BLK, 16), blk),
            pl.BlockSpec((13 * D, D), lambda i: (0, 0)),
            pl.BlockSpec((D,), lambda i: (0,)),
        ],
        out_specs=pl.BlockSpec((_ROW_BLK, D), blk),
        out_shape=jax.ShapeDtypeStruct((N, D), jnp.float32),
    )(h, s, mx, mn, sq, deg, w, b)


def _sc_agg_body(h_hbm, src_hbm, dst_hbm, offs_hbm,
                 o_s, o_mx, o_mn, o_sq, o_deg,
                 mx_v, mn_v, s_a, sq_a, deg_v,
                 rows0, rows1, src0, src1, dloc0, dloc1, offs_v,
                 sem_g0, sem_g1, sem_i0, sem_i1):
    cid = lax.axis_index("c")
    sid = lax.axis_index("s")
    wid = sid * 2 + cid

    iota = lax.iota(jnp.int32, 16)
    zeros16 = jnp.zeros((16,), jnp.float32)
    negv = jnp.full((16,), NEG, jnp.float32)
    posv = jnp.full((16,), POS, jnp.float32)
    ones16 = jnp.ones((16,), jnp.float32)
    cols = [iota + 16 * k for k in range(D // 16)]
    rows = (rows0, rows1)
    srcs = (src0, src1)
    dlocs = (dloc0, dloc1)
    sems_g = (sem_g0, sem_g1)
    sems_i = (sem_i0, sem_i1)

    pltpu.sync_copy(offs_hbm, offs_v)

    for wave in range(NWAVES):
        slot = wave * NW + wid
        base_node = slot * NPT

        def _init(i, _):
            for k in range(D // 16):
                mx_v[i, pl.ds(16 * k, 16)] = negv
                mn_v[i, pl.ds(16 * k, 16)] = posv
                s_a[i, pl.ds(16 * k, 16)] = zeros16
                sq_a[i, pl.ds(16 * k, 16)] = zeros16
            deg_v[pl.ds(16 * i, 16)] = zeros16
            return 0
        lax.fori_loop(0, ARWS, _init, 0)

        sltv = jnp.full((16,), slot, jnp.int32)
        start = lax.reduce_max(plsc.load_gather(offs_v, [sltv]), (0,))
        end = lax.reduce_max(plsc.load_gather(offs_v, [sltv + 1]), (0,))
        astart = start & ~7
        nchunks = (end - astart + (C - 1)) // C
        npairs = (nchunks + 1) >> 1

        def _cbase(c):
            return pl.multiple_of(astart + c * C, 8)

        def _grp(c, dl):
            cbase = _cbase(c)
            for g in range(C // 16):
                ids = jnp.full((16,), cbase + g * 16, jnp.int32) + iota
                valid = (ids >= start) & (ids < end)
                loc = dl[pl.ds(g * 16, 16)] - base_node
                dl[pl.ds(g * 16, 16)] = jnp.where(valid, loc, GARB)

        def _issue_idx(c, p):
            cbase = _cbase(c)
            pltpu.async_copy(src_hbm.at[pl.ds(cbase, C)], srcs[p], sems_i[p])
            pltpu.async_copy(dst_hbm.at[pl.ds(cbase, C)], dlocs[p], sems_i[p])

        def _drain_idx(p):
            pltpu.make_async_copy(src_hbm.at[pl.ds(0, C)], srcs[p],
                                  sems_i[p]).wait()
            pltpu.make_async_copy(dst_hbm.at[pl.ds(0, C)], dlocs[p],
                                  sems_i[p]).wait()

        def _issue_gather(p):
            pltpu.async_copy(h_hbm.at[srcs[p]], rows[p], sems_g[p])

        def _drain_gather(p):
            pltpu.make_async_copy(h_hbm.at[pl.ds(0, C)], rows[p],
                                  sems_g[p]).wait()

        def _edges(p):
            rp = rows[p]
            dp = dlocs[p]

            fours16 = jnp.full((16,), 4.0, jnp.float32)

            def _edge(jj, _):
                jb = jj * 4
                bv = [plsc.load_gather(dp, [jnp.full((16,), jb + u,
                                                     jnp.int32)])
                      for u in range(4)]
                mm = [[rp[jb + u, pl.ds(16 * k, 16)]
                       for k in range(D // 16)] for u in range(4)]
                same = ((bv[0] == bv[1]) & (bv[1] == bv[2])
                        & (bv[2] == bv[3]))
                allsame = jnp.all(same)

                @pl.when(allsame)
                def _fast():
                    b = bv[0]
                    cmx = [plsc.load_gather(mx_v, [b, cols[k]])
                           for k in range(D // 16)]
                    cmn = [plsc.load_gather(mn_v, [b, cols[k]])
                           for k in range(D // 16)]
                    for k in range(D // 16):
                        m0, m1, m2, m3 = (mm[0][k], mm[1][k], mm[2][k],
                                          mm[3][k])
                        gmx = jnp.maximum(jnp.maximum(m0, m1),
                                          jnp.maximum(m2, m3))
                        gmn = jnp.minimum(jnp.minimum(m0, m1),
                                          jnp.minimum(m2, m3))
                        gs = (m0 + m1) + (m2 + m3)
                        gq = (m0 * m0 + m1 * m1) + (m2 * m2 + m3 * m3)
                        plsc.store_scatter(mx_v, [b, cols[k]],
                                           jnp.maximum(cmx[k], gmx))
                        plsc.store_scatter(mn_v, [b, cols[k]],
                                           jnp.minimum(cmn[k], gmn))
                        plsc.addupdate_scatter(s_a, [b, cols[k]], gs)
                        plsc.addupdate_scatter(sq_a, [b, cols[k]], gq)
                    plsc.addupdate_scatter(deg_v, [b * 16 + iota], fours16)

                @pl.when(jnp.logical_not(allsame))
                def _slow():
                    for u in range(4):
                        bvec = bv[u]
                        dgi = bvec * 16 + iota
                        ms = mm[u]
                        cmx = [plsc.load_gather(mx_v, [bvec, cols[k]])
                               for k in range(D // 16)]
                        cmn = [plsc.load_gather(mn_v, [bvec, cols[k]])
                               for k in range(D // 16)]
                        for k in range(D // 16):
                            plsc.store_scatter(mx_v, [bvec, cols[k]],
                                               jnp.maximum(cmx[k], ms[k]))
                            plsc.store_scatter(mn_v, [bvec, cols[k]],
                                               jnp.minimum(cmn[k], ms[k]))
                        for k in range(D // 16):
                            plsc.addupdate_scatter(s_a, [bvec, cols[k]],
                                                   ms[k])
                            plsc.addupdate_scatter(sq_a, [bvec, cols[k]],
                                                   ms[k] * ms[k])
                        plsc.addupdate_scatter(deg_v, [dgi], ones16)
                return 0
            lax.fori_loop(0, C // 4, _edge, 0)

        a0 = pl.multiple_of(astart, 8)
        pltpu.sync_copy(src_hbm.at[pl.ds(a0, C)], src0)
        pltpu.sync_copy(dst_hbm.at[pl.ds(a0, C)], dloc0)
        _grp(0, dloc0)
        _issue_gather(0)
        _issue_idx(1, 1)

        def _chunk(c, p):
            q = 1 - p
            _drain_idx(q)
            _grp(c + 1, dlocs[q])
            _issue_gather(q)
            _drain_gather(p)
            _edges(p)
            _issue_idx(c + 2, p)

        def _pair(i, _):
            _chunk(2 * i, 0)
            _chunk(2 * i + 1, 1)
            return 0
        lax.fori_loop(0, npairs, _pair, 0)

        _drain_gather(0)
        _drain_idx(1)

        pltpu.sync_copy(mx_v.at[pl.ds(0, NPT)],
                        o_mx.at[pl.ds(base_node, NPT)])
        pltpu.sync_copy(mn_v.at[pl.ds(0, NPT)],
                        o_mn.at[pl.ds(base_node, NPT)])
        pltpu.sync_copy(s_a.at[pl.ds(0, NPT)],
                        o_s.at[pl.ds(base_node, NPT)])
        pltpu.sync_copy(sq_a.at[pl.ds(0, NPT)],
                        o_sq.at[pl.ds(base_node, NPT)])
        pltpu.sync_copy(deg_v.at[pl.ds(0, NPT * 16)],
                        o_deg.at[pl.ds(pl.multiple_of(base_node * 16, 8),
                                       NPT * 16)])


def _sc_aggregate(h, src_pad, dst_pad, offs):
    mesh = plsc.VectorSubcoreMesh(core_axis_name="c", subcore_axis_name="s")
    f32 = jnp.float32
    out_type = [
        jax.ShapeDtypeStruct((NPAD, D), f32),
        jax.ShapeDtypeStruct((NPAD, D), f32),
        jax.ShapeDtypeStruct((NPAD, D), f32),
        jax.ShapeDtypeStruct((NPAD, D), f32),
        jax.ShapeDtypeStruct((NPAD * 16,), f32),
    ]
    scratch = [
        pltpu.VMEM((ARWS, D), f32),
        pltpu.VMEM((ARWS, D), f32),
        pltpu.VMEM((ARWS, D), f32),
        pltpu.VMEM((ARWS, D), f32),
        pltpu.VMEM((ARWS * 16,), f32),
        pltpu.VMEM((C, D), f32),
        pltpu.VMEM((C, D), f32),
        pltpu.VMEM((C,), jnp.int32),
        pltpu.VMEM((C,), jnp.int32),
        pltpu.VMEM((C,), jnp.int32),
        pltpu.VMEM((C,), jnp.int32),
        pltpu.VMEM((NSLOT + 8,), jnp.int32),
        pltpu.SemaphoreType.DMA,
        pltpu.SemaphoreType.DMA,
        pltpu.SemaphoreType.DMA,
        pltpu.SemaphoreType.DMA,
    ]
    kern = pl.kernel(_sc_agg_body, out_type=out_type, mesh=mesh,
                     scratch_types=scratch,
                     compiler_params=pltpu.CompilerParams(
                         needs_layout_passes=False))
    return kern(h, src_pad, dst_pad, offs)


def kernel(h, edge_index, e, W_enc, b_enc, W0, b0, W1, b1, W2, b2, W3, b3,
           W_ro, b_ro):
    src = edge_index[0].astype(jnp.int32)
    dst = edge_index[1].astype(jnp.int32)
    dst_s, src_s = lax.sort((dst, src), num_keys=1)
    pad = jnp.zeros((4 * C,), jnp.int32)
    src_pad = jnp.concatenate([src_s, pad])
    dst_pad = jnp.concatenate([dst_s, pad])
    bounds = jnp.arange(0, NPAD + 1, NPT, dtype=jnp.int32)
    offs = jnp.searchsorted(dst_s, bounds).astype(jnp.int32)
    offs = jnp.concatenate([offs, jnp.full((7,), E, jnp.int32)])

    h = _matmul_bias(h, W_enc, b_enc)
    for W, b in ((W0, b0), (W1, b1), (W2, b2), (W3, b3)):
        s, mx, mn, sq, deg16 = _sc_aggregate(h, src_pad, dst_pad, offs)
        h = _layer_combine(h, s, mx, mn, sq, deg16.reshape(NPAD, 16), W, b)
    return _matmul_bias(h, W_ro, b_ro)


# speedup vs baseline: 9.4708x; 1.0203x over previous
import functools

import jax
import jax.numpy as jnp
from jax import lax
from jax.experimental import pallas as pl
from jax.experimental.pallas import tpu as pltpu
from jax.experimental.pallas import tpu_sc as plsc

N = 10000
D = 128
E = 320000
AVG_D_LOG = 3.5

_ROW_BLK = 1000

NPT = 160
NW = 32
NWAVES = 2
NSLOT = NW * NWAVES
NPAD = NPT * NSLOT
GARB = NPT
ARWS = 168
C = 128
NEG = -3.0e38
POS = 3.0e38


def _mm_kernel(x_ref, w_ref, b_ref, o_ref, *, relu):
    acc = jnp.dot(x_ref[...], w_ref[...], preferred_element_type=jnp.float32)
    acc = acc + b_ref[...][None, :]
    if relu:
        acc = jnp.maximum(acc, 0.0)
    o_ref[...] = acc


def _matmul_bias(x, w, b, relu=False):
    n, k = x.shape
    m = w.shape[1]
    grid = (n // _ROW_BLK,)
    return pl.pallas_call(
        functools.partial(_mm_kernel, relu=relu),
        grid=grid,
        in_specs=[
            pl.BlockSpec((_ROW_BLK, k), lambda i: (i, 0)),
            pl.BlockSpec((k, m), lambda i: (0, 0)),
            pl.BlockSpec((m,), lambda i: (0,)),
        ],
        out_specs=pl.BlockSpec((_ROW_BLK, m), lambda i: (i, 0)),
        out_shape=jax.ShapeDtypeStruct((n, m), jnp.float32),
    )(x, w, b)


def _combine_kernel(h_ref, s_ref, mx_ref, mn_ref, sq_ref, deg_ref, w_ref,
                    b_ref, o_ref):
    deg = deg_ref[...][:, 0:1]
    degc = jnp.maximum(deg, 1.0)
    invd = 1.0 / degc
    has = deg > 0.0
    mean = s_ref[...] * invd
    msq = sq_ref[...] * invd
    std = jnp.sqrt(jnp.maximum(msq - mean * mean, 0.0) + 1e-5)
    mx = jnp.where(has, mx_ref[...], 0.0)
    mn = jnp.where(has, mn_ref[...], 0.0)
    agg = jnp.concatenate([mean, mx, mn, std], axis=1)
    ld = jnp.log(deg + 1.0)
    amp = ld / AVG_D_LOG
    att = AVG_D_LOG / jnp.maximum(ld, 1e-5)
    w = w_ref[...]
    acc = jnp.dot(h_ref[...], w[0:D], preferred_element_type=jnp.float32)
    acc += jnp.dot(agg, w[D:D + 512], preferred_element_type=jnp.float32)
    acc += jnp.dot(agg * amp, w[D + 512:D + 1024],
                   preferred_element_type=jnp.float32)
    acc += jnp.dot(agg * att, w[D + 1024:D + 1536],
                   preferred_element_type=jnp.float32)
    acc += b_ref[...][None, :]
    o_ref[...] = jnp.maximum(acc, 0.0)


def _layer_combine(h, s, mx, mn, sq, deg, w, b):
    grid = (N // _ROW_BLK,)
    blk = lambda i: (i, 0)
    return pl.pallas_call(
        _combine_kernel,
        grid=grid,
        in_specs=[
            pl.BlockSpec((_ROW_BLK, D), blk),
            pl.BlockSpec((_ROW_BLK, D), blk),
            pl.BlockSpec((_ROW_BLK, D), blk),
            pl.BlockSpec((_ROW_BLK, D), blk),
            pl.BlockSpec((_ROW_BLK, D), blk),
            pl.BlockSpec((_ROW_BLK, 16), blk),
            pl.BlockSpec((13 * D, D), lambda i: (0, 0)),
            pl.BlockSpec((D,), lambda i: (0,)),
        ],
        out_specs=pl.BlockSpec((_ROW_BLK, D), blk),
        out_shape=jax.ShapeDtypeStruct((N, D), jnp.float32),
    )(h, s, mx, mn, sq, deg, w, b)


def _sc_agg_body(h_hbm, src_hbm, dst_hbm, offs_hbm,
                 o_s, o_mx, o_mn, o_sq, o_deg,
                 mx_v, mn_v, s_a, sq_a, deg_v,
                 rows0, rows1, src0, src1, dloc0, dloc1, offs_v,
                 sem_g0, sem_g1, sem_i0, sem_i1):
    cid = lax.axis_index("c")
    sid = lax.axis_index("s")
    wid = sid * 2 + cid

    iota = lax.iota(jnp.int32, 16)
    zeros16 = jnp.zeros((16,), jnp.float32)
    negv = jnp.full((16,), NEG, jnp.float32)
    posv = jnp.full((16,), POS, jnp.float32)
    ones16 = jnp.ones((16,), jnp.float32)
    cols = [iota + 16 * k for k in range(D // 16)]
    rows = (rows0, rows1)
    srcs = (src0, src1)
    dlocs = (dloc0, dloc1)
    sems_g = (sem_g0, sem_g1)
    sems_i = (sem_i0, sem_i1)

    pltpu.sync_copy(offs_hbm, offs_v)

    for wave in range(NWAVES):
        slot = wave * NW + wid
        base_node = slot * NPT

        def _init(i, _):
            for k in range(D // 16):
                mx_v[i, pl.ds(16 * k, 16)] = negv
                mn_v[i, pl.ds(16 * k, 16)] = posv
                s_a[i, pl.ds(16 * k, 16)] = zeros16
                sq_a[i, pl.ds(16 * k, 16)] = zeros16
            deg_v[pl.ds(16 * i, 16)] = zeros16
            return 0
        lax.fori_loop(0, ARWS, _init, 0)

        sltv = jnp.full((16,), slot, jnp.int32)
        start = lax.reduce_max(plsc.load_gather(offs_v, [sltv]), (0,))
        end = lax.reduce_max(plsc.load_gather(offs_v, [sltv + 1]), (0,))
        astart = start & ~7
        nchunks = (end - astart + (C - 1)) // C
        npairs = (nchunks + 1) >> 1

        def _cbase(c):
            return pl.multiple_of(astart + c * C, 8)

        def _grp(c, dl):
            cbase = _cbase(c)
            for g in range(C // 16):
                ids = jnp.full((16,), cbase + g * 16, jnp.int32) + iota
                valid = (ids >= start) & (ids < end)
                loc = dl[pl.ds(g * 16, 16)] - base_node
                dl[pl.ds(g * 16, 16)] = jnp.where(valid, loc, GARB)

        def _issue_idx(c, p):
            cbase = _cbase(c)
            pltpu.async_copy(src_hbm.at[pl.ds(cbase, C)], srcs[p], sems_i[p])
            pltpu.async_copy(dst_hbm.at[pl.ds(cbase, C)], dlocs[p], sems_i[p])

        def _drain_idx(p):
            pltpu.make_async_copy(src_hbm.at[pl.ds(0, C)], srcs[p],
                                  sems_i[p]).wait()
            pltpu.make_async_copy(dst_hbm.at[pl.ds(0, C)], dlocs[p],
                                  sems_i[p]).wait()

        def _issue_gather(p):
            pltpu.async_copy(h_hbm.at[srcs[p]], rows[p], sems_g[p])

        def _drain_gather(p):
            pltpu.make_async_copy(h_hbm.at[pl.ds(0, C)], rows[p],
                                  sems_g[p]).wait()

        def _edges(p):
            rp = rows[p]
            dp = dlocs[p]

            fours16 = jnp.full((16,), 4.0, jnp.float32)
            eights16 = jnp.full((16,), 8.0, jnp.float32)

            def _rmw4(b, g4):
                cmx = [plsc.load_gather(mx_v, [b, cols[k]])
                       for k in range(D // 16)]
                cmn = [plsc.load_gather(mn_v, [b, cols[k]])
                       for k in range(D // 16)]
                for k in range(D // 16):
                    m0, m1, m2, m3 = g4[k]
                    gmx = jnp.maximum(jnp.maximum(m0, m1),
                                      jnp.maximum(m2, m3))
                    gmn = jnp.minimum(jnp.minimum(m0, m1),
                                      jnp.minimum(m2, m3))
                    gs = (m0 + m1) + (m2 + m3)
                    gq = (m0 * m0 + m1 * m1) + (m2 * m2 + m3 * m3)
                    plsc.store_scatter(mx_v, [b, cols[k]],
                                       jnp.maximum(cmx[k], gmx))
                    plsc.store_scatter(mn_v, [b, cols[k]],
                                       jnp.minimum(cmn[k], gmn))
                    plsc.addupdate_scatter(s_a, [b, cols[k]], gs)
                    plsc.addupdate_scatter(sq_a, [b, cols[k]], gq)
                plsc.addupdate_scatter(deg_v, [b * 16 + iota], fours16)

            def _edge1(bvec, ms):
                cmx = [plsc.load_gather(mx_v, [bvec, cols[k]])
                       for k in range(D // 16)]
                cmn = [plsc.load_gather(mn_v, [bvec, cols[k]])
                       for k in range(D // 16)]
                for k in range(D // 16):
                    plsc.store_scatter(mx_v, [bvec, cols[k]],
                                       jnp.maximum(cmx[k], ms[k]))
                    plsc.store_scatter(mn_v, [bvec, cols[k]],
                                       jnp.minimum(cmn[k], ms[k]))
                for k in range(D // 16):
                    plsc.addupdate_scatter(s_a, [bvec, cols[k]], ms[k])
                    plsc.addupdate_scatter(sq_a, [bvec, cols[k]],
                                           ms[k] * ms[k])
                plsc.addupdate_scatter(deg_v, [bvec * 16 + iota], ones16)

            def _do4(bv, mm):
                same = ((bv[0] == bv[1]) & (bv[1] == bv[2])
                        & (bv[2] == bv[3]))
                allsame = jnp.all(same)

                @pl.when(allsame)
                def _fast():
                    _rmw4(bv[0], [(mm[0][k], mm[1][k], mm[2][k], mm[3][k])
                                  for k in range(D // 16)])

                @pl.when(jnp.logical_not(allsame))
                def _slow():
                    for u in range(4):
                        _edge1(bv[u], mm[u])

            def _edge(jj, _):
                jb = jj * 8
                bv = [plsc.load_gather(dp, [jnp.full((16,), jb + u,
                                                     jnp.int32)])
                      for u in range(8)]
                mm = [[rp[jb + u, pl.ds(16 * k, 16)]
                       for k in range(D // 16)] for u in range(8)]
                same8 = ((bv[0] == bv[1]) & (bv[1] == bv[2])
                         & (bv[2] == bv[3]) & (bv[3] == bv[4])
                         & (bv[4] == bv[5]) & (bv[5] == bv[6])
                         & (bv[6] == bv[7]))
                all8 = jnp.all(same8)

                @pl.when(all8)
                def _fast8():
                    b = bv[0]
                    cmx = [plsc.load_gather(mx_v, [b, cols[k]])
                           for k in range(D // 16)]
                    cmn = [plsc.load_gather(mn_v, [b, cols[k]])
                           for k in range(D // 16)]
                    for k in range(D // 16):
                        m = [mm[u][k] for u in range(8)]
                        x01 = jnp.maximum(m[0], m[1])
                        x23 = jnp.maximum(m[2], m[3])
                        x45 = jnp.maximum(m[4], m[5])
                        x67 = jnp.maximum(m[6], m[7])
                        gmx = jnp.maximum(jnp.maximum(x01, x23),
                                          jnp.maximum(x45, x67))
                        n01 = jnp.minimum(m[0], m[1])
                        n23 = jnp.minimum(m[2], m[3])
                        n45 = jnp.minimum(m[4], m[5])
                        n67 = jnp.minimum(m[6], m[7])
                        gmn = jnp.minimum(jnp.minimum(n01, n23),
                                          jnp.minimum(n45, n67))
                        gs = ((m[0] + m[1]) + (m[2] + m[3])) + \
                             ((m[4] + m[5]) + (m[6] + m[7]))
                        gq = ((m[0] * m[0] + m[1] * m[1])
                              + (m[2] * m[2] + m[3] * m[3])) + \
                             ((m[4] * m[4] + m[5] * m[5])
                              + (m[6] * m[6] + m[7] * m[7]))
                        plsc.store_scatter(mx_v, [b, cols[k]],
                                           jnp.maximum(cmx[k], gmx))
                        plsc.store_scatter(mn_v, [b, cols[k]],
                                           jnp.minimum(cmn[k], gmn))
                        plsc.addupdate_scatter(s_a, [b, cols[k]], gs)
                        plsc.addupdate_scatter(sq_a, [b, cols[k]], gq)
                    plsc.addupdate_scatter(deg_v, [b * 16 + iota], eights16)

                @pl.when(jnp.logical_not(all8))
                def _split():
                    _do4(bv[0:4], mm[0:4])
                    _do4(bv[4:8], mm[4:8])
                return 0
            lax.fori_loop(0, C // 8, _edge, 0)

        a0 = pl.multiple_of(astart, 8)
        pltpu.sync_copy(src_hbm.at[pl.ds(a0, C)], src0)
        pltpu.sync_copy(dst_hbm.at[pl.ds(a0, C)], dloc0)
        _grp(0, dloc0)
        _issue_gather(0)
        _issue_idx(1, 1)

        def _chunk(c, p):
            q = 1 - p
            _drain_idx(q)
            _grp(c + 1, dlocs[q])
            _issue_gather(q)
            _drain_gather(p)
            _edges(p)
            _issue_idx(c + 2, p)

        def _pair(i, _):
            _chunk(2 * i, 0)
            _chunk(2 * i + 1, 1)
            return 0
        lax.fori_loop(0, npairs, _pair, 0)

        _drain_gather(0)
        _drain_idx(1)

        pltpu.sync_copy(mx_v.at[pl.ds(0, NPT)],
                        o_mx.at[pl.ds(base_node, NPT)])
        pltpu.sync_copy(mn_v.at[pl.ds(0, NPT)],
                        o_mn.at[pl.ds(base_node, NPT)])
        pltpu.sync_copy(s_a.at[pl.ds(0, NPT)],
                        o_s.at[pl.ds(base_node, NPT)])
        pltpu.sync_copy(sq_a.at[pl.ds(0, NPT)],
                        o_sq.at[pl.ds(base_node, NPT)])
        pltpu.sync_copy(deg_v.at[pl.ds(0, NPT * 16)],
                        o_deg.at[pl.ds(pl.multiple_of(base_node * 16, 8),
                                       NPT * 16)])


def _sc_aggregate(h, src_pad, dst_pad, offs):
    mesh = plsc.VectorSubcoreMesh(core_axis_name="c", subcore_axis_name="s")
    f32 = jnp.float32
    out_type = [
        jax.ShapeDtypeStruct((NPAD, D), f32),
        jax.ShapeDtypeStruct((NPAD, D), f32),
        jax.ShapeDtypeStruct((NPAD, D), f32),
        jax.ShapeDtypeStruct((NPAD, D), f32),
        jax.ShapeDtypeStruct((NPAD * 16,), f32),
    ]
    scratch = [
        pltpu.VMEM((ARWS, D), f32),
        pltpu.VMEM((ARWS, D), f32),
        pltpu.VMEM((ARWS, D), f32),
        pltpu.VMEM((ARWS, D), f32),
        pltpu.VMEM((ARWS * 16,), f32),
        pltpu.VMEM((C, D), f32),
        pltpu.VMEM((C, D), f32),
        pltpu.VMEM((C,), jnp.int32),
        pltpu.VMEM((C,), jnp.int32),
        pltpu.VMEM((C,), jnp.int32),
        pltpu.VMEM((C,), jnp.int32),
        pltpu.VMEM((NSLOT + 8,), jnp.int32),
        pltpu.SemaphoreType.DMA,
        pltpu.SemaphoreType.DMA,
        pltpu.SemaphoreType.DMA,
        pltpu.SemaphoreType.DMA,
    ]
    kern = pl.kernel(_sc_agg_body, out_type=out_type, mesh=mesh,
                     scratch_types=scratch,
                     compiler_params=pltpu.CompilerParams(
                         needs_layout_passes=False))
    return kern(h, src_pad, dst_pad, offs)


def kernel(h, edge_index, e, W_enc, b_enc, W0, b0, W1, b1, W2, b2, W3, b3,
           W_ro, b_ro):
    src = edge_index[0].astype(jnp.int32)
    dst = edge_index[1].astype(jnp.int32)
    dst_s, src_s = lax.sort((dst, src), num_keys=1)
    pad = jnp.zeros((4 * C,), jnp.int32)
    src_pad = jnp.concatenate([src_s, pad])
    dst_pad = jnp.concatenate([dst_s, pad])
    bounds = jnp.arange(0, NPAD + 1, NPT, dtype=jnp.int32)
    offs = jnp.searchsorted(dst_s, bounds).astype(jnp.int32)
    offs = jnp.concatenate([offs, jnp.full((7,), E, jnp.int32)])

    h = _matmul_bias(h, W_enc, b_enc)
    for W, b in ((W0, b0), (W1, b1), (W2, b2), (W3, b3)):
        s, mx, mn, sq, deg16 = _sc_aggregate(h, src_pad, dst_pad, offs)
        h = _layer_combine(h, s, mx, mn, sq, deg16.reshape(NPAD, 16), W, b)
    return _matmul_bias(h, W_ro, b_ro)
